# Initial kernel scaffold; baseline (speedup 1.0000x reference)
import jax
import jax.numpy as jnp
from jax.experimental import pallas as pl


def kernel(state, value):
    raise NotImplementedError("write your pallas kernel here")


# speedup vs baseline: 1.1129x; 1.1129x over previous
import functools

import jax
import jax.numpy as jnp
from jax import lax
from jax.experimental import pallas as pl
from jax.experimental.pallas import tpu as pltpu
from jax.experimental.pallas import tpu_sc as plsc


def _gather_call(batch: int):
    info = plsc.get_sparse_core_info()
    nc, ns = info.num_cores, info.num_subcores
    nw = nc * ns
    bpw = batch // nw
    mesh = plsc.VectorSubcoreMesh(core_axis_name="c", subcore_axis_name="s")

    @functools.partial(
        pl.kernel,
        mesh=mesh,
        out_type=jax.ShapeDtypeStruct((batch,), jnp.float32),
        scratch_types=[
            pltpu.VMEM((bpw,), jnp.int32),
            pltpu.VMEM((bpw,), jnp.float32),
            pltpu.SemaphoreType.DMA,
        ],
    )
    def gather_k(value_hbm, state_hbm, out_hbm, idx_v, vals_v, sem):
        wid = lax.axis_index("s") * nc + lax.axis_index("c")
        base = wid * bpw
        pltpu.sync_copy(state_hbm.at[pl.ds(base, bpw)], idx_v)
        pltpu.async_copy(value_hbm.at[idx_v], vals_v, sem).wait()
        pltpu.sync_copy(vals_v, out_hbm.at[pl.ds(base, bpw)])

    return gather_k


def kernel(state, value):
    state = state.astype(jnp.int32)
    return _gather_call(state.shape[0])(value, state)


# speedup vs baseline: 1.1147x; 1.0016x over previous
import functools

import jax
import jax.numpy as jnp
from jax import lax
from jax.experimental import pallas as pl
from jax.experimental.pallas import tpu as pltpu
from jax.experimental.pallas import tpu_sc as plsc

_NCHUNK = 4


def _gather_call(batch: int):
    info = plsc.get_sparse_core_info()
    nc, ns = info.num_cores, info.num_subcores
    nw = nc * ns
    bpw = batch // nw
    ch = bpw // _NCHUNK
    mesh = plsc.VectorSubcoreMesh(core_axis_name="c", subcore_axis_name="s")

    @functools.partial(
        pl.kernel,
        mesh=mesh,
        out_type=jax.ShapeDtypeStruct((batch,), jnp.float32),
        scratch_types=[
            pltpu.VMEM((bpw,), jnp.int32),
            pltpu.VMEM((bpw,), jnp.float32),
            pltpu.SemaphoreType.DMA((_NCHUNK,)),
            pltpu.SemaphoreType.DMA((_NCHUNK,)),
            pltpu.SemaphoreType.DMA((_NCHUNK,)),
        ],
    )
    def gather_k(value_hbm, state_hbm, out_hbm, idx_v, vals_v, isem, gsem, wsem):
        wid = lax.axis_index("s") * nc + lax.axis_index("c")
        base = wid * bpw
        loads = [
            pltpu.async_copy(
                state_hbm.at[pl.ds(base + j * ch, ch)],
                idx_v.at[pl.ds(j * ch, ch)],
                isem.at[j],
            )
            for j in range(_NCHUNK)
        ]
        gathers = []
        for j in range(_NCHUNK):
            loads[j].wait()
            gathers.append(
                pltpu.async_copy(
                    value_hbm.at[idx_v.at[pl.ds(j * ch, ch)]],
                    vals_v.at[pl.ds(j * ch, ch)],
                    gsem.at[j],
                )
            )
        writes = []
        for j in range(_NCHUNK):
            gathers[j].wait()
            writes.append(
                pltpu.async_copy(
                    vals_v.at[pl.ds(j * ch, ch)],
                    out_hbm.at[pl.ds(base + j * ch, ch)],
                    wsem.at[j],
                )
            )
        for w in writes:
            w.wait()

    return gather_k


def kernel(state, value):
    state = state.astype(jnp.int32)
    return _gather_call(state.shape[0])(value, state)


# speedup vs baseline: 1.1716x; 1.0510x over previous
import functools

import jax
import jax.numpy as jnp
from jax import lax
from jax.experimental import pallas as pl
from jax.experimental.pallas import tpu as pltpu
from jax.experimental.pallas import tpu_sc as plsc

_NCHUNK = 4


def _gather_call(batch: int):
    info = plsc.get_sparse_core_info()
    nc, ns = 1, info.num_subcores
    nw = nc * ns
    bpw = batch // nw
    ch = bpw // _NCHUNK
    mesh = plsc.VectorSubcoreMesh(core_axis_name="c", subcore_axis_name="s", num_cores=1)

    @functools.partial(
        pl.kernel,
        mesh=mesh,
        out_type=jax.ShapeDtypeStruct((batch,), jnp.float32),
        scratch_types=[
            pltpu.VMEM((bpw,), jnp.int32),
            pltpu.VMEM((bpw,), jnp.float32),
            pltpu.SemaphoreType.DMA((_NCHUNK,)),
            pltpu.SemaphoreType.DMA((_NCHUNK,)),
            pltpu.SemaphoreType.DMA((_NCHUNK,)),
        ],
    )
    def gather_k(value_hbm, state_hbm, out_hbm, idx_v, vals_v, isem, gsem, wsem):
        wid = lax.axis_index("s") * nc + lax.axis_index("c")
        base = wid * bpw
        loads = [
            pltpu.async_copy(
                state_hbm.at[pl.ds(base + j * ch, ch)],
                idx_v.at[pl.ds(j * ch, ch)],
                isem.at[j],
            )
            for j in range(_NCHUNK)
        ]
        gathers = []
        for j in range(_NCHUNK):
            loads[j].wait()
            gathers.append(
                pltpu.async_copy(
                    value_hbm.at[idx_v.at[pl.ds(j * ch, ch)]],
                    vals_v.at[pl.ds(j * ch, ch)],
                    gsem.at[j],
                )
            )
        writes = []
        for j in range(_NCHUNK):
            gathers[j].wait()
            writes.append(
                pltpu.async_copy(
                    vals_v.at[pl.ds(j * ch, ch)],
                    out_hbm.at[pl.ds(base + j * ch, ch)],
                    wsem.at[j],
                )
            )
        for w in writes:
            w.wait()

    return gather_k


def kernel(state, value):
    state = state.astype(jnp.int32)
    return _gather_call(state.shape[0])(value, state)
